# baseline (speedup 1.0000x reference)
import functools
import jax
import jax.numpy as jnp
from jax import lax
from jax.experimental import pallas as pl
from jax.experimental.pallas import tpu as pltpu
from jax.experimental.pallas import tpu_sc as plsc


@functools.lru_cache(maxsize=None)
def _build(B, V, D):
    info = plsc.get_sparse_core_info()
    NC, NS = info.num_cores, info.num_subcores
    NW = NC * NS
    b_per_w = B // NW
    mesh = plsc.VectorSubcoreMesh(core_axis_name="c", subcore_axis_name="s")

    @functools.partial(
        pl.kernel,
        mesh=mesh,
        compiler_params=pltpu.CompilerParams(use_tc_tiling_on_sc=True, needs_layout_passes=True),
        out_type=jax.ShapeDtypeStruct((B, D), jnp.float32),
        scratch_types=[
            pltpu.VMEM((b_per_w, D), jnp.float32),
        ],
    )
    def k(idx_hbm, table_hbm, out_hbm, rows_v):
        wid = lax.axis_index("s") * NC + lax.axis_index("c")
        pltpu.sync_copy(table_hbm.at[pl.ds(wid * b_per_w, b_per_w)], rows_v)
        pltpu.sync_copy(rows_v, out_hbm.at[pl.ds(wid * b_per_w, b_per_w)])

    return k


def kernel(user_idx, table):
    B, = user_idx.shape
    V, D = table.shape
    return _build(B, V, D)(user_idx.astype(jnp.int32), table)


# speedup vs baseline: 2.1951x; 2.1951x over previous
import functools

import jax
import jax.numpy as jnp
from jax import lax
from jax.experimental import pallas as pl
from jax.experimental.pallas import tpu as pltpu
from jax.experimental.pallas import tpu_sc as plsc

_L = 16
_G = 8


@functools.lru_cache(maxsize=None)
def _build(B, V, D):
    info = plsc.get_sparse_core_info()
    NC, NS = info.num_cores, info.num_subcores
    NW = NC * NS
    b_per_w = B // NW
    n_dg = b_per_w // _L
    n_band = D // 8
    rows_per_idx = n_band * 8

    mesh = plsc.VectorSubcoreMesh(core_axis_name="c", subcore_axis_name="s")

    @functools.partial(
        pl.kernel,
        mesh=mesh,
        compiler_params=pltpu.CompilerParams(use_tc_tiling_on_sc=True, needs_layout_passes=False),
        out_type=jax.ShapeDtypeStruct((B, D), jnp.float32),
        scratch_types=[
            pltpu.VMEM((b_per_w,), jnp.int32),
            pltpu.VMEM((2 * _G * rows_per_idx, 128), jnp.float32),
            pltpu.VMEM((2, _G, D), jnp.float32),
            pltpu.SemaphoreType.DMA,
            pltpu.SemaphoreType.DMA,
            pltpu.SemaphoreType.DMA,
            pltpu.SemaphoreType.DMA,
        ],
    )
    def gather_kernel(
        idx_hbm, table_hbm, out_hbm, idx_v, tile_v, rows_v,
        sem_a, sem_b, osem_a, osem_b,
    ):
        wid = lax.axis_index("s") * NC + lax.axis_index("c")
        base = wid * b_per_w
        pltpu.sync_copy(idx_hbm.at[wid], idx_v)

        iota = lax.iota(jnp.int32, _L)

        tsems = (sem_a, sem_b)
        osems = (osem_a, osem_b)

        def fire(vv, half):
            for j in range(_G):
                i_scalar = vv[half * _G + j]
                cbase = pl.multiple_of((i_scalar >> 7) * 128, 128)
                slot = (half * _G + j) * rows_per_idx
                for b in range(n_band):
                    pltpu.async_copy(
                        table_hbm.at[pl.ds(8 * b, 8), pl.ds(cbase, 128)],
                        tile_v.at[pl.ds(slot + 8 * b, 8)],
                        tsems[half],
                    )

        def drain_tiles(half):
            for _ in range(_G * n_band):
                pltpu.make_async_copy(
                    table_hbm.at[pl.ds(0, 8), pl.ds(0, 128)],
                    tile_v.at[pl.ds(0, 8)],
                    tsems[half],
                ).wait()

        def extract(vv, half):
            for j in range(_G):
                i_scalar = vv[half * _G + j]
                l_vec = jnp.full((_L,), i_scalar & 127, jnp.int32)
                slot = (half * _G + j) * rows_per_idx
                r0 = iota + slot
                r1 = iota + (slot + _L)
                h0 = plsc.load_gather(tile_v, [r0, l_vec])
                h1 = plsc.load_gather(tile_v, [r1, l_vec])
                rows_v[half, j, pl.ds(0, _L)] = h0
                rows_v[half, j, pl.ds(_L, _L)] = h1

        def flush(g, half):
            pltpu.async_copy(
                rows_v.at[half],
                out_hbm.at[pl.ds(base + g * _G, _G)],
                osems[half],
            )

        def drain_flush(half):
            pltpu.make_async_copy(
                rows_v.at[half], out_hbm.at[pl.ds(0, _G)], osems[half]
            ).wait()

        vv0 = idx_v[pl.ds(0, _L)]
        fire(vv0, 0)

        def body(gg, _):
            vv = idx_v[pl.ds(gg * _L, _L)]
            off2 = jnp.minimum((gg + 1) * _L, (n_dg - 1) * _L)
            vv2 = idx_v[pl.ds(pl.multiple_of(off2, _L), _L)]
            fire(vv, 1)
            drain_tiles(0)
            lax.cond(gg > 0, lambda: drain_flush(0), lambda: None)
            extract(vv, 0)
            flush(2 * gg, 0)
            fire(vv2, 0)
            drain_tiles(1)
            lax.cond(gg > 0, lambda: drain_flush(1), lambda: None)
            extract(vv, 1)
            flush(2 * gg + 1, 1)
            return 0

        lax.fori_loop(0, n_dg, body, 0)
        drain_tiles(0)
        drain_flush(0)
        drain_flush(1)

    return gather_kernel


def kernel(user_idx, table):
    B, = user_idx.shape
    V, D = table.shape
    info = plsc.get_sparse_core_info()
    NW = info.num_cores * info.num_subcores
    idx = user_idx.astype(jnp.int32).reshape(NW, -1)
    return _build(B, V, D)(idx, table.T)


# speedup vs baseline: 2.2197x; 1.0112x over previous
import functools

import jax
import jax.numpy as jnp
from jax import lax
from jax.experimental import pallas as pl
from jax.experimental.pallas import tpu as pltpu
from jax.experimental.pallas import tpu_sc as plsc

_L = 16
_G = 8


@functools.lru_cache(maxsize=None)
def _build(B, V, D):
    info = plsc.get_sparse_core_info()
    NC, NS = info.num_cores, info.num_subcores
    NW = NC * NS
    b_per_w = B // NW
    n_dg = b_per_w // _L
    n_band = D // 8
    rows_per_idx = n_band * 8

    mesh = plsc.VectorSubcoreMesh(core_axis_name="c", subcore_axis_name="s")

    @functools.partial(
        pl.kernel,
        mesh=mesh,
        compiler_params=pltpu.CompilerParams(use_tc_tiling_on_sc=True, needs_layout_passes=False),
        out_type=jax.ShapeDtypeStruct((B, D), jnp.float32),
        scratch_types=[
            pltpu.VMEM((b_per_w,), jnp.int32),
            pltpu.VMEM((2 * _G * rows_per_idx, 128), jnp.float32),
            pltpu.VMEM((2, _G, D), jnp.float32),
            pltpu.SemaphoreType.DMA,
            pltpu.SemaphoreType.DMA,
            pltpu.SemaphoreType.DMA,
            pltpu.SemaphoreType.DMA,
        ],
    )
    def gather_kernel(
        idx_hbm, table_hbm, out_hbm, idx_v, tile_v, rows_v,
        sem_a, sem_b, osem_a, osem_b,
    ):
        wid = lax.axis_index("s") * NC + lax.axis_index("c")
        base = wid * b_per_w
        pltpu.sync_copy(idx_hbm.at[wid], idx_v)

        iota = lax.iota(jnp.int32, _L)

        tsems = (sem_a, sem_b)
        osems = (osem_a, osem_b)

        def fire(vv, half):
            for j in range(_G):
                i_scalar = vv[half * _G + j]
                cbase = pl.multiple_of((i_scalar >> 7) * 128, 128)
                slot = (half * _G + j) * rows_per_idx
                pltpu.async_copy(
                    table_hbm.at[:, pl.ds(cbase, 128)],
                    tile_v.at[pl.ds(slot, rows_per_idx)],
                    tsems[half],
                )

        def drain_tiles(half):
            for _ in range(_G):
                pltpu.make_async_copy(
                    table_hbm.at[:, pl.ds(0, 128)],
                    tile_v.at[pl.ds(0, rows_per_idx)],
                    tsems[half],
                ).wait()

        def extract(vv, half):
            for j in range(_G):
                i_scalar = vv[half * _G + j]
                l_vec = jnp.full((_L,), i_scalar & 127, jnp.int32)
                slot = (half * _G + j) * rows_per_idx
                r0 = iota + slot
                r1 = iota + (slot + _L)
                h0 = plsc.load_gather(tile_v, [r0, l_vec])
                h1 = plsc.load_gather(tile_v, [r1, l_vec])
                rows_v[half, j, pl.ds(0, _L)] = h0
                rows_v[half, j, pl.ds(_L, _L)] = h1

        def flush(g, half):
            pltpu.async_copy(
                rows_v.at[half],
                out_hbm.at[pl.ds(base + g * _G, _G)],
                osems[half],
            )

        def drain_flush(half):
            pltpu.make_async_copy(
                rows_v.at[half], out_hbm.at[pl.ds(0, _G)], osems[half]
            ).wait()

        vv0 = idx_v[pl.ds(0, _L)]
        fire(vv0, 0)

        def body(gg, _):
            vv = idx_v[pl.ds(gg * _L, _L)]
            off2 = jnp.minimum((gg + 1) * _L, (n_dg - 1) * _L)
            vv2 = idx_v[pl.ds(pl.multiple_of(off2, _L), _L)]
            fire(vv, 1)
            drain_tiles(0)
            lax.cond(gg > 0, lambda: drain_flush(0), lambda: None)
            extract(vv, 0)
            flush(2 * gg, 0)
            fire(vv2, 0)
            drain_tiles(1)
            lax.cond(gg > 0, lambda: drain_flush(1), lambda: None)
            extract(vv, 1)
            flush(2 * gg + 1, 1)
            return 0

        lax.fori_loop(0, n_dg, body, 0)
        drain_tiles(0)
        drain_flush(0)
        drain_flush(1)

    return gather_kernel


def kernel(user_idx, table):
    B, = user_idx.shape
    V, D = table.shape
    info = plsc.get_sparse_core_info()
    NW = info.num_cores * info.num_subcores
    idx = user_idx.astype(jnp.int32).reshape(NW, -1)
    return _build(B, V, D)(idx, table.T)
